# baseline (speedup 1.0000x reference)
import jax
import jax.numpy as jnp
import numpy as np
from jax.experimental import pallas as pl
from jax.experimental.pallas import tpu as pltpu

_N_KERNELS = 29
_LANES = 128


def _mmd_body(xs_ref, xt_ref, aux_ref, out_ref):
    xs = xs_ref[...]
    xt = xt_ref[...]
    xs_n = pltpu.roll(xs, 7, 1)
    xt_n = pltpu.roll(xt, 7, 1)

    def sqd(u, v):
        d = u - v
        return jnp.sum(d * d, axis=2, keepdims=True)

    dxx = sqd(xs, xs_n)
    dyy = sqd(xt, xt_n)
    dxy = sqd(xs, xt_n)
    dyx = sqd(xt, xs_n)

    c = aux_ref[0:1, :].reshape(1, 1, _LANES)
    w = aux_ref[1:2, :].reshape(1, 1, _LANES)
    s = (jnp.exp(dxx * c) + jnp.exp(dyy * c)
         - jnp.exp(dxy * c) - jnp.exp(dyx * c))

    rows = jax.lax.broadcasted_iota(jnp.int32, s.shape, 1)
    even = (rows % 2) == 0
    part = jnp.sum(jnp.where(even, s * w, 0.0))
    out_ref[...] = jnp.full_like(out_ref, part)


def kernel(Xs, Xt, betas):
    n, f = Xs.shape
    m = (n // 8) * 8
    p = m // 2

    gammas = np.power(np.float32(2.0),
                      np.arange(-3.5, 3.75, 0.25, dtype=np.float32))
    neg_inv = (-1.0 / (2.0 * gammas * gammas)).astype(np.float32)
    aux = jnp.zeros((8, _LANES), dtype=jnp.float32)
    aux = aux.at[0, :_N_KERNELS].set(jnp.asarray(neg_inv))
    aux = aux.at[1, :_N_KERNELS].set(betas[:, 0] / np.float32(p))

    b8 = 512
    ng = m // (b8 * 8)
    assert ng * b8 * 8 == m, (m, ng, b8)

    out = pl.pallas_call(
        _mmd_body,
        grid=(ng,),
        in_specs=[
            pl.BlockSpec((b8, 8, f), lambda i: (i, 0, 0)),
            pl.BlockSpec((b8, 8, f), lambda i: (i, 0, 0)),
            pl.BlockSpec((8, _LANES), lambda i: (0, 0)),
        ],
        out_specs=pl.BlockSpec((1, 1, _LANES), lambda i: (i, 0, 0)),
        out_shape=jax.ShapeDtypeStruct((ng, 1, _LANES), jnp.float32),
        compiler_params=pltpu.CompilerParams(
            dimension_semantics=("parallel",),
        ),
    )(Xs[:m].reshape(m // 8, 8, f), Xt[:m].reshape(m // 8, 8, f), aux)

    return jnp.sum(out[:, 0, 0]).reshape(1)
